# baseline (speedup 1.0000x reference)
import functools

import jax
import jax.numpy as jnp
from jax.experimental import pallas as pl
from jax.experimental.pallas import tpu as pltpu

N_WORDS = 8192
N_HEADS = 512
IN = 1024
HID = 1024
MID = 256
DE = 64
MAXD = 128
T = 256
HALO = 8
W = T + 2 * HALO
NT = N_WORDS // T


def _mm_kernel(a_ref, b_ref, o_ref):
    o_ref[...] = jnp.dot(a_ref[...], b_ref[...],
                         preferred_element_type=jnp.float32)


def _wp_matmul(words, w1b):
    return pl.pallas_call(
        _mm_kernel,
        grid=(16, 2),
        in_specs=[pl.BlockSpec((512, IN), lambda i, j: (i, 0)),
                  pl.BlockSpec((IN, 512), lambda i, j: (0, j))],
        out_specs=pl.BlockSpec((512, 512), lambda i, j: (i, j)),
        out_shape=jax.ShapeDtypeStruct((N_WORDS, HID), jnp.float32),
    )(words, w1b)


def _e1_matmul(emb, w1c):
    return pl.pallas_call(
        _mm_kernel,
        in_specs=[pl.BlockSpec((MAXD, DE), lambda: (0, 0)),
                  pl.BlockSpec((DE, HID), lambda: (0, 0))],
        out_specs=pl.BlockSpec((MAXD, HID), lambda: (0, 0)),
        out_shape=jax.ShapeDtypeStruct((MAXD, HID), jnp.float32),
    )(emb, w1c)


def _hw_kernel(hids_ref, wrow_ref, w1a_ref, b1_ref, o_ref):
    o_ref[0] = (jnp.dot(wrow_ref[0], w1a_ref[...],
                        preferred_element_type=jnp.float32)
                + b1_ref[...])


def _hw_matmul(heads_ids, words3, w1a, b1):
    grid_spec = pltpu.PrefetchScalarGridSpec(
        num_scalar_prefetch=1,
        grid=(N_HEADS,),
        in_specs=[pl.BlockSpec((1, 1, IN), lambda h, hids: (hids[h], 0, 0)),
                  pl.BlockSpec((IN, HID), lambda h, hids: (0, 0)),
                  pl.BlockSpec((1, HID), lambda h, hids: (0, 0))],
        out_specs=pl.BlockSpec((1, 1, HID), lambda h, hids: (h, 0, 0)),
    )
    return pl.pallas_call(
        _hw_kernel,
        grid_spec=grid_spec,
        out_shape=jax.ShapeDtypeStruct((N_HEADS, 1, HID), jnp.float32),
    )(heads_ids, words3, w1a, b1)


def _span_kernel(starts_ref, lens_ref, hids_ref, maxl_ref,
                 wpp_ref, hw_ref, e1_ref, w2_ref, b2_ref, w3_ref, b3_ref,
                 a1_ref, bc1_ref, a2_ref, bc2_ref,
                 out_ref, wp_scr, sem):
    h = pl.program_id(0)
    t = pl.program_id(1)
    start = starts_ref[h]
    length = lens_ref[h]
    head_id = hids_ref[h]
    maxl = maxl_ref[0]
    tile0 = t * T
    active = jnp.logical_and(start < tile0 + T, start + length > tile0)

    @pl.when(jnp.logical_not(active))
    def _():
        out_ref[...] = jnp.full((1, T, 2), -jnp.inf, jnp.float32)

    @pl.when(active)
    def _():
        cp = pltpu.make_async_copy(wpp_ref.at[pl.ds(tile0, W), :], wp_scr, sem)
        cp.start()

        ii = jax.lax.broadcasted_iota(jnp.int32, (W, 1), 0)
        p = tile0 - HALO + ii
        k = p - start

        e = head_id + (MAXD - 2) // 2 - p
        idx = jnp.where((e < 0) | (e > MAXD - 2), MAXD - 1, e)
        col = jax.lax.broadcasted_iota(jnp.int32, (W, MAXD), 1)
        oh = (col == idx).astype(jnp.float32)
        embc = jnp.dot(oh, e1_ref[...], preferred_element_type=jnp.float32)

        cp.wait()
        h1 = jnp.maximum(wp_scr[...] + hw_ref[0] + embc, 0.0)
        h2 = jnp.maximum(
            jnp.dot(h1, w2_ref[...], preferred_element_type=jnp.float32)
            + b2_ref[...], 0.0)
        h3 = jnp.dot(h2, w3_ref[...], preferred_element_type=jnp.float32) \
            + b3_ref[...]
        h3 = jnp.where((k >= 0) & (k < length), h3, 0.0)

        x1 = (jnp.dot(h3[0:W - 2], a1_ref[0], preferred_element_type=jnp.float32)
              + jnp.dot(h3[1:W - 1], a1_ref[1], preferred_element_type=jnp.float32)
              + jnp.dot(h3[2:W], a1_ref[2], preferred_element_type=jnp.float32)
              + bc1_ref[...])
        x1 = jnp.concatenate(
            [jnp.zeros((1, 4), jnp.float32), x1, jnp.zeros((1, 4), jnp.float32)],
            axis=0)
        x1 = jnp.where((k >= 0) & (k < maxl), x1, 0.0)

        res = (jnp.dot(x1[0:W - 2], a2_ref[0], preferred_element_type=jnp.float32)
               + jnp.dot(x1[1:W - 1], a2_ref[1], preferred_element_type=jnp.float32)
               + jnp.dot(x1[2:W], a2_ref[2], preferred_element_type=jnp.float32)
               + bc2_ref[...])
        vals = res[HALO - 1:HALO - 1 + T]
        jj = jax.lax.broadcasted_iota(jnp.int32, (T, 1), 0)
        kq = tile0 + jj - start
        valid = (kq >= 0) & (kq < length)
        out_ref[...] = jnp.where(valid, vals, -jnp.inf)[None]


def kernel(sent_id, words, heads_ids, W1, b1, W2, b2, W3, b3, emb_table,
           Wc1, bc1, Wc2, bc2):
    heads_ids = heads_ids.astype(jnp.int32)
    head_sent = sent_id[heads_ids]
    starts = jnp.searchsorted(sent_id, head_sent, side='left').astype(jnp.int32)
    ends = jnp.searchsorted(sent_id, head_sent, side='right').astype(jnp.int32)
    lengths = ends - starts
    maxl = jnp.max(lengths).reshape(1)

    w1a = W1[:IN]
    w1b = W1[IN:2 * IN]
    w1c = W1[2 * IN:]

    wp = _wp_matmul(words, w1b)
    wpp = jnp.pad(wp, ((HALO, HALO), (0, 0)))
    hw = _hw_matmul(heads_ids, words.reshape(N_WORDS, 1, IN), w1a,
                    b1.reshape(1, HID))
    e1 = _e1_matmul(emb_table, w1c)

    a1 = jnp.transpose(Wc1, (2, 1, 0))
    a2 = jnp.transpose(Wc2, (2, 1, 0))

    grid_spec = pltpu.PrefetchScalarGridSpec(
        num_scalar_prefetch=4,
        grid=(N_HEADS, NT),
        in_specs=[
            pl.BlockSpec(memory_space=pl.ANY),
            pl.BlockSpec((1, 1, HID), lambda h, t, *_: (h, 0, 0)),
            pl.BlockSpec((MAXD, HID), lambda h, t, *_: (0, 0)),
            pl.BlockSpec((HID, MID), lambda h, t, *_: (0, 0)),
            pl.BlockSpec((1, MID), lambda h, t, *_: (0, 0)),
            pl.BlockSpec((MID, DE), lambda h, t, *_: (0, 0)),
            pl.BlockSpec((1, DE), lambda h, t, *_: (0, 0)),
            pl.BlockSpec((3, DE, 4), lambda h, t, *_: (0, 0, 0)),
            pl.BlockSpec((1, 4), lambda h, t, *_: (0, 0)),
            pl.BlockSpec((3, 4, 2), lambda h, t, *_: (0, 0, 0)),
            pl.BlockSpec((1, 2), lambda h, t, *_: (0, 0)),
        ],
        out_specs=pl.BlockSpec((1, T, 2), lambda h, t, *_: (h, t, 0)),
        scratch_shapes=[pltpu.VMEM((W, HID), jnp.float32),
                        pltpu.SemaphoreType.DMA],
    )
    return pl.pallas_call(
        _span_kernel,
        grid_spec=grid_spec,
        out_shape=jax.ShapeDtypeStruct((N_HEADS, N_WORDS, 2), jnp.float32),
        compiler_params=pltpu.CompilerParams(
            dimension_semantics=("arbitrary", "arbitrary")),
    )(starts, lengths, heads_ids, maxl,
      wpp, hw, e1, W2, b2.reshape(1, MID), W3, b3.reshape(1, DE),
      a1, bc1.reshape(1, 4), a2, bc2.reshape(1, 2))


# speedup vs baseline: 1.3420x; 1.3420x over previous
import functools

import jax
import jax.numpy as jnp
from jax.experimental import pallas as pl
from jax.experimental.pallas import tpu as pltpu

N_WORDS = 8192
N_HEADS = 512
IN = 1024
HID = 1024
MID = 256
DE = 64
MAXD = 128
T = 512
HALO = 8
W = T + 2 * HALO
NT = N_WORDS // T


def _mm_kernel(a_ref, b_ref, o_ref):
    o_ref[...] = jnp.dot(a_ref[...], b_ref[...],
                         preferred_element_type=jnp.float32)


def _wp_matmul(words, w1b):
    return pl.pallas_call(
        _mm_kernel,
        grid=(16, 2),
        in_specs=[pl.BlockSpec((512, IN), lambda i, j: (i, 0)),
                  pl.BlockSpec((IN, 512), lambda i, j: (0, j))],
        out_specs=pl.BlockSpec((512, 512), lambda i, j: (i, j)),
        out_shape=jax.ShapeDtypeStruct((N_WORDS, HID), jnp.float32),
    )(words, w1b)


def _e1_matmul(emb, w1c):
    return pl.pallas_call(
        _mm_kernel,
        in_specs=[pl.BlockSpec((MAXD, DE), lambda: (0, 0)),
                  pl.BlockSpec((DE, HID), lambda: (0, 0))],
        out_specs=pl.BlockSpec((MAXD, HID), lambda: (0, 0)),
        out_shape=jax.ShapeDtypeStruct((MAXD, HID), jnp.float32),
    )(emb, w1c)


def _hw_kernel(hids_ref, wrow_ref, w1a_ref, b1_ref, o_ref):
    o_ref[0] = (jnp.dot(wrow_ref[0], w1a_ref[...],
                        preferred_element_type=jnp.float32)
                + b1_ref[...])


def _hw_matmul(heads_ids, words3, w1a, b1):
    grid_spec = pltpu.PrefetchScalarGridSpec(
        num_scalar_prefetch=1,
        grid=(N_HEADS,),
        in_specs=[pl.BlockSpec((1, 1, IN), lambda h, hids: (hids[h], 0, 0)),
                  pl.BlockSpec((IN, HID), lambda h, hids: (0, 0)),
                  pl.BlockSpec((1, HID), lambda h, hids: (0, 0))],
        out_specs=pl.BlockSpec((1, 1, HID), lambda h, hids: (h, 0, 0)),
    )
    return pl.pallas_call(
        _hw_kernel,
        grid_spec=grid_spec,
        out_shape=jax.ShapeDtypeStruct((N_HEADS, 1, HID), jnp.float32),
    )(heads_ids, words3, w1a, b1)


def _span_kernel(starts_ref, lens_ref, hids_ref, maxl_ref,
                 wpp_ref, hw_ref, e1_ref, w2_ref, b2_ref, w3_ref, b3_ref,
                 a1_ref, bc1_ref, a2_ref, bc2_ref,
                 out_ref, wp_scr, sem):
    h = pl.program_id(0)
    t = pl.program_id(1)
    start = starts_ref[h]
    length = lens_ref[h]
    head_id = hids_ref[h]
    maxl = maxl_ref[0]
    tile0 = t * T
    active = jnp.logical_and(start < tile0 + T, start + length > tile0)

    @pl.when(jnp.logical_not(active))
    def _():
        out_ref[...] = jnp.full((1, T, 2), -jnp.inf, jnp.float32)

    @pl.when(active)
    def _():
        cp = pltpu.make_async_copy(wpp_ref.at[pl.ds(tile0, W), :], wp_scr, sem)
        cp.start()

        ii = jax.lax.broadcasted_iota(jnp.int32, (W, 1), 0)
        p = tile0 - HALO + ii
        k = p - start

        e = head_id + (MAXD - 2) // 2 - p
        idx = jnp.where((e < 0) | (e > MAXD - 2), MAXD - 1, e)
        col = jax.lax.broadcasted_iota(jnp.int32, (W, MAXD), 1)
        oh = (col == idx).astype(jnp.bfloat16)
        embc = jnp.dot(oh, e1_ref[...], preferred_element_type=jnp.float32)

        cp.wait()
        h1 = jnp.maximum(wp_scr[...] + hw_ref[0] + embc, 0.0)
        h2 = jnp.maximum(
            jnp.dot(h1.astype(jnp.bfloat16), w2_ref[...],
                    preferred_element_type=jnp.float32)
            + b2_ref[...], 0.0)
        h3 = jnp.dot(h2, w3_ref[...], preferred_element_type=jnp.float32) \
            + b3_ref[...]
        h3 = jnp.where((k >= 0) & (k < length), h3, 0.0)

        x1 = (jnp.dot(h3[0:W - 2], a1_ref[0], preferred_element_type=jnp.float32)
              + jnp.dot(h3[1:W - 1], a1_ref[1], preferred_element_type=jnp.float32)
              + jnp.dot(h3[2:W], a1_ref[2], preferred_element_type=jnp.float32)
              + bc1_ref[...])
        x1 = jnp.concatenate(
            [jnp.zeros((1, 4), jnp.float32), x1, jnp.zeros((1, 4), jnp.float32)],
            axis=0)
        x1 = jnp.where((k >= 0) & (k < maxl), x1, 0.0)

        res = (jnp.dot(x1[0:W - 2], a2_ref[0], preferred_element_type=jnp.float32)
               + jnp.dot(x1[1:W - 1], a2_ref[1], preferred_element_type=jnp.float32)
               + jnp.dot(x1[2:W], a2_ref[2], preferred_element_type=jnp.float32)
               + bc2_ref[...])
        vals = res[HALO - 1:HALO - 1 + T]
        jj = jax.lax.broadcasted_iota(jnp.int32, (T, 1), 0)
        kq = tile0 + jj - start
        valid = (kq >= 0) & (kq < length)
        out_ref[...] = jnp.where(valid, vals, -jnp.inf)[None]


def kernel(sent_id, words, heads_ids, W1, b1, W2, b2, W3, b3, emb_table,
           Wc1, bc1, Wc2, bc2):
    heads_ids = heads_ids.astype(jnp.int32)
    head_sent = sent_id[heads_ids]
    starts = jnp.searchsorted(sent_id, head_sent, side='left').astype(jnp.int32)
    ends = jnp.searchsorted(sent_id, head_sent, side='right').astype(jnp.int32)
    lengths = ends - starts
    maxl = jnp.max(lengths).reshape(1)

    w1a = W1[:IN]
    w1b = W1[IN:2 * IN]
    w1c = W1[2 * IN:]

    wp = _wp_matmul(words, w1b)
    wpp = jnp.pad(wp, ((HALO, HALO), (0, 0)))
    hw = _hw_matmul(heads_ids, words.reshape(N_WORDS, 1, IN), w1a,
                    b1.reshape(1, HID))
    e1 = _e1_matmul(emb_table, w1c)

    a1 = jnp.transpose(Wc1, (2, 1, 0))
    a2 = jnp.transpose(Wc2, (2, 1, 0))

    grid_spec = pltpu.PrefetchScalarGridSpec(
        num_scalar_prefetch=4,
        grid=(N_HEADS, NT),
        in_specs=[
            pl.BlockSpec(memory_space=pl.ANY),
            pl.BlockSpec((1, 1, HID), lambda h, t, *_: (h, 0, 0)),
            pl.BlockSpec((MAXD, HID), lambda h, t, *_: (0, 0)),
            pl.BlockSpec((HID, MID), lambda h, t, *_: (0, 0)),
            pl.BlockSpec((1, MID), lambda h, t, *_: (0, 0)),
            pl.BlockSpec((MID, DE), lambda h, t, *_: (0, 0)),
            pl.BlockSpec((1, DE), lambda h, t, *_: (0, 0)),
            pl.BlockSpec((3, DE, 4), lambda h, t, *_: (0, 0, 0)),
            pl.BlockSpec((1, 4), lambda h, t, *_: (0, 0)),
            pl.BlockSpec((3, 4, 2), lambda h, t, *_: (0, 0, 0)),
            pl.BlockSpec((1, 2), lambda h, t, *_: (0, 0)),
        ],
        out_specs=pl.BlockSpec((1, T, 2), lambda h, t, *_: (h, t, 0)),
        scratch_shapes=[pltpu.VMEM((W, HID), jnp.float32),
                        pltpu.SemaphoreType.DMA],
    )
    return pl.pallas_call(
        _span_kernel,
        grid_spec=grid_spec,
        out_shape=jax.ShapeDtypeStruct((N_HEADS, N_WORDS, 2), jnp.float32),
        compiler_params=pltpu.CompilerParams(
            dimension_semantics=("arbitrary", "arbitrary")),
    )(starts, lengths, heads_ids, maxl,
      wpp, hw, e1.astype(jnp.bfloat16), W2.astype(jnp.bfloat16),
      b2.reshape(1, MID), W3, b3.reshape(1, DE),
      a1, bc1.reshape(1, 4), a2, bc2.reshape(1, 2))


# speedup vs baseline: 1.8087x; 1.3478x over previous
import functools

import jax
import jax.numpy as jnp
from jax.experimental import pallas as pl
from jax.experimental.pallas import tpu as pltpu

N_WORDS = 8192
N_HEADS = 512
IN = 1024
HID = 1024
MID = 256
DE = 64
MAXD = 128
T = 512
HALO = 8
W = T + 2 * HALO
NT = N_WORDS // T


def _mm_kernel(a_ref, b_ref, o_ref):
    o_ref[...] = jnp.dot(a_ref[...], b_ref[...],
                         preferred_element_type=jnp.float32)


def _wp_matmul(words, w1b):
    return pl.pallas_call(
        _mm_kernel,
        grid=(16, 2),
        in_specs=[pl.BlockSpec((512, IN), lambda i, j: (i, 0)),
                  pl.BlockSpec((IN, 512), lambda i, j: (0, j))],
        out_specs=pl.BlockSpec((512, 512), lambda i, j: (i, j)),
        out_shape=jax.ShapeDtypeStruct((N_WORDS, HID), jnp.float32),
    )(words, w1b)


def _e1_matmul(emb, w1c):
    return pl.pallas_call(
        _mm_kernel,
        in_specs=[pl.BlockSpec((MAXD, DE), lambda: (0, 0)),
                  pl.BlockSpec((DE, HID), lambda: (0, 0))],
        out_specs=pl.BlockSpec((MAXD, HID), lambda: (0, 0)),
        out_shape=jax.ShapeDtypeStruct((MAXD, HID), jnp.float32),
    )(emb, w1c)


def _hw_kernel(hids_ref, wrow_ref, w1a_ref, b1_ref, o_ref):
    o_ref[0] = (jnp.dot(wrow_ref[0], w1a_ref[...],
                        preferred_element_type=jnp.float32)
                + b1_ref[...])


def _hw_matmul(heads_ids, words3, w1a, b1):
    grid_spec = pltpu.PrefetchScalarGridSpec(
        num_scalar_prefetch=1,
        grid=(N_HEADS,),
        in_specs=[pl.BlockSpec((1, 1, IN), lambda h, hids: (hids[h], 0, 0)),
                  pl.BlockSpec((IN, HID), lambda h, hids: (0, 0)),
                  pl.BlockSpec((1, HID), lambda h, hids: (0, 0))],
        out_specs=pl.BlockSpec((1, 1, HID), lambda h, hids: (h, 0, 0)),
    )
    return pl.pallas_call(
        _hw_kernel,
        grid_spec=grid_spec,
        out_shape=jax.ShapeDtypeStruct((N_HEADS, 1, HID), jnp.float32),
    )(heads_ids, words3, w1a, b1)


def _span_kernel(starts_ref, lens_ref, hids_ref, maxl_ref,
                 wpp_ref, hw_ref, e1_ref, w2_ref, b2_ref, w3_ref, b3_ref,
                 a1_ref, bc1_ref, a2_ref, bc2_ref,
                 out_ref, wp_scr, sem):
    h = pl.program_id(0)
    t = pl.program_id(1)
    start = starts_ref[h]
    length = lens_ref[h]
    head_id = hids_ref[h]
    maxl = maxl_ref[0]
    tile0 = t * T
    active = jnp.logical_and(start < tile0 + T, start + length > tile0)

    @pl.when(jnp.logical_not(active))
    def _():
        out_ref[...] = jnp.full((1, 2, T), -jnp.inf, jnp.float32)

    @pl.when(active)
    def _():
        cp = pltpu.make_async_copy(wpp_ref.at[pl.ds(tile0, W), :], wp_scr, sem)
        cp.start()

        ii = jax.lax.broadcasted_iota(jnp.int32, (W, 1), 0)
        p = tile0 - HALO + ii
        k = p - start

        e = head_id + (MAXD - 2) // 2 - p
        idx = jnp.where((e < 0) | (e > MAXD - 2), MAXD - 1, e)
        col = jax.lax.broadcasted_iota(jnp.int32, (W, MAXD), 1)
        oh = (col == idx).astype(jnp.float32)
        embc = jnp.dot(oh, e1_ref[...], preferred_element_type=jnp.float32)

        cp.wait()
        h1 = jnp.maximum(wp_scr[...] + hw_ref[0] + embc, 0.0)
        h2 = jnp.maximum(
            jnp.dot(h1, w2_ref[...], preferred_element_type=jnp.float32)
            + b2_ref[...], 0.0)
        h3 = jnp.dot(h2, w3_ref[...], preferred_element_type=jnp.float32) \
            + b3_ref[...]
        h3 = jnp.where((k >= 0) & (k < length), h3, 0.0)

        x1 = (jnp.dot(h3[0:W - 2], a1_ref[0], preferred_element_type=jnp.float32)
              + jnp.dot(h3[1:W - 1], a1_ref[1], preferred_element_type=jnp.float32)
              + jnp.dot(h3[2:W], a1_ref[2], preferred_element_type=jnp.float32)
              + bc1_ref[...])
        x1 = jnp.concatenate(
            [jnp.zeros((1, 4), jnp.float32), x1, jnp.zeros((1, 4), jnp.float32)],
            axis=0)
        x1 = jnp.where((k >= 0) & (k < maxl), x1, 0.0)

        res = (jnp.dot(x1[0:W - 2], a2_ref[0], preferred_element_type=jnp.float32)
               + jnp.dot(x1[1:W - 1], a2_ref[1], preferred_element_type=jnp.float32)
               + jnp.dot(x1[2:W], a2_ref[2], preferred_element_type=jnp.float32)
               + bc2_ref[...])
        vals = jnp.transpose(res[HALO - 1:HALO - 1 + T])
        jj = jax.lax.broadcasted_iota(jnp.int32, (1, T), 1)
        kq = tile0 + jj - start
        valid = (kq >= 0) & (kq < length)
        out_ref[...] = jnp.where(valid, vals, -jnp.inf)[None]


def kernel(sent_id, words, heads_ids, W1, b1, W2, b2, W3, b3, emb_table,
           Wc1, bc1, Wc2, bc2):
    heads_ids = heads_ids.astype(jnp.int32)
    head_sent = sent_id[heads_ids]
    starts = jnp.searchsorted(sent_id, head_sent, side='left').astype(jnp.int32)
    ends = jnp.searchsorted(sent_id, head_sent, side='right').astype(jnp.int32)
    lengths = ends - starts
    maxl = jnp.max(lengths).reshape(1)

    w1a = W1[:IN]
    w1b = W1[IN:2 * IN]
    w1c = W1[2 * IN:]

    wp = _wp_matmul(words, w1b)
    wpp = jnp.pad(wp, ((HALO, HALO), (0, 0)))
    hw = _hw_matmul(heads_ids, words.reshape(N_WORDS, 1, IN), w1a,
                    b1.reshape(1, HID))
    e1 = _e1_matmul(emb_table, w1c)

    a1 = jnp.transpose(Wc1, (2, 1, 0))
    a2 = jnp.transpose(Wc2, (2, 1, 0))

    grid_spec = pltpu.PrefetchScalarGridSpec(
        num_scalar_prefetch=4,
        grid=(N_HEADS, NT),
        in_specs=[
            pl.BlockSpec(memory_space=pl.ANY),
            pl.BlockSpec((1, 1, HID), lambda h, t, *_: (h, 0, 0)),
            pl.BlockSpec((MAXD, HID), lambda h, t, *_: (0, 0)),
            pl.BlockSpec((HID, MID), lambda h, t, *_: (0, 0)),
            pl.BlockSpec((1, MID), lambda h, t, *_: (0, 0)),
            pl.BlockSpec((MID, DE), lambda h, t, *_: (0, 0)),
            pl.BlockSpec((1, DE), lambda h, t, *_: (0, 0)),
            pl.BlockSpec((3, DE, 4), lambda h, t, *_: (0, 0, 0)),
            pl.BlockSpec((1, 4), lambda h, t, *_: (0, 0)),
            pl.BlockSpec((3, 4, 2), lambda h, t, *_: (0, 0, 0)),
            pl.BlockSpec((1, 2), lambda h, t, *_: (0, 0)),
        ],
        out_specs=pl.BlockSpec((1, 2, T), lambda h, t, *_: (h, 0, t)),
        scratch_shapes=[pltpu.VMEM((W, HID), jnp.float32),
                        pltpu.SemaphoreType.DMA],
    )
    out = pl.pallas_call(
        _span_kernel,
        grid_spec=grid_spec,
        out_shape=jax.ShapeDtypeStruct((N_HEADS, 2, N_WORDS), jnp.float32),
        compiler_params=pltpu.CompilerParams(
            dimension_semantics=("arbitrary", "arbitrary")),
    )(starts, lengths, heads_ids, maxl,
      wpp, hw, e1, W2,
      b2.reshape(1, MID), W3, b3.reshape(1, DE),
      a1, bc1.reshape(1, 4), a2, bc2.reshape(1, 2))
    return jnp.transpose(out, (0, 2, 1))


# speedup vs baseline: 2.0112x; 1.1120x over previous
import functools

import jax
import jax.numpy as jnp
from jax.experimental import pallas as pl
from jax.experimental.pallas import tpu as pltpu

N_WORDS = 8192
N_HEADS = 512
IN = 1024
HID = 1024
MID = 256
DE = 64
MAXD = 128
T = 1024
HALO = 8
W = T + 2 * HALO
NT = N_WORDS // T


def _mm_kernel(a_ref, b_ref, o_ref):
    o_ref[...] = jnp.dot(a_ref[...], b_ref[...],
                         preferred_element_type=jnp.float32)


def _wp_matmul(words, w1b):
    return pl.pallas_call(
        _mm_kernel,
        grid=(16, 2),
        in_specs=[pl.BlockSpec((512, IN), lambda i, j: (i, 0)),
                  pl.BlockSpec((IN, 512), lambda i, j: (0, j))],
        out_specs=pl.BlockSpec((512, 512), lambda i, j: (i, j)),
        out_shape=jax.ShapeDtypeStruct((N_WORDS, HID), jnp.float32),
    )(words, w1b)


def _e1_matmul(emb, w1c):
    return pl.pallas_call(
        _mm_kernel,
        in_specs=[pl.BlockSpec((MAXD, DE), lambda: (0, 0)),
                  pl.BlockSpec((DE, HID), lambda: (0, 0))],
        out_specs=pl.BlockSpec((MAXD, HID), lambda: (0, 0)),
        out_shape=jax.ShapeDtypeStruct((MAXD, HID), jnp.float32),
    )(emb, w1c)


def _hw_kernel(hids_ref, wrow_ref, w1a_ref, b1_ref, o_ref):
    o_ref[0] = (jnp.dot(wrow_ref[0], w1a_ref[...],
                        preferred_element_type=jnp.float32)
                + b1_ref[...])


def _hw_matmul(heads_ids, words3, w1a, b1):
    grid_spec = pltpu.PrefetchScalarGridSpec(
        num_scalar_prefetch=1,
        grid=(N_HEADS,),
        in_specs=[pl.BlockSpec((1, 1, IN), lambda h, hids: (hids[h], 0, 0)),
                  pl.BlockSpec((IN, HID), lambda h, hids: (0, 0)),
                  pl.BlockSpec((1, HID), lambda h, hids: (0, 0))],
        out_specs=pl.BlockSpec((1, 1, HID), lambda h, hids: (h, 0, 0)),
    )
    return pl.pallas_call(
        _hw_kernel,
        grid_spec=grid_spec,
        out_shape=jax.ShapeDtypeStruct((N_HEADS, 1, HID), jnp.float32),
    )(heads_ids, words3, w1a, b1)


def _span_kernel(starts_ref, lens_ref, hids_ref, maxl_ref,
                 wpp_ref, hw_ref, e1_ref, w2_ref, b2_ref, w3_ref, b3_ref,
                 a1_ref, bc1_ref, a2_ref, bc2_ref,
                 out_ref, wp_scr, sem):
    h = pl.program_id(0)
    t = pl.program_id(1)
    start = starts_ref[h]
    length = lens_ref[h]
    head_id = hids_ref[h]
    maxl = maxl_ref[0]
    tile0 = t * T
    active = jnp.logical_and(start < tile0 + T, start + length > tile0)

    @pl.when(jnp.logical_not(active))
    def _():
        out_ref[...] = jnp.full((1, 2, T), -jnp.inf, jnp.float32)

    @pl.when(active)
    def _():
        cp = pltpu.make_async_copy(wpp_ref.at[pl.ds(tile0, W), :], wp_scr, sem)
        cp.start()

        ii = jax.lax.broadcasted_iota(jnp.int32, (W, 1), 0)
        p = tile0 - HALO + ii
        k = p - start

        e = head_id + (MAXD - 2) // 2 - p
        idx = jnp.where((e < 0) | (e > MAXD - 2), MAXD - 1, e)
        col = jax.lax.broadcasted_iota(jnp.int32, (W, MAXD), 1)
        oh = (col == idx).astype(jnp.float32)
        embc = jnp.dot(oh, e1_ref[...], preferred_element_type=jnp.float32)

        cp.wait()
        h1 = jnp.maximum(wp_scr[...] + hw_ref[0] + embc, 0.0)
        h2 = jnp.maximum(
            jnp.dot(h1, w2_ref[...], preferred_element_type=jnp.float32)
            + b2_ref[...], 0.0)
        h3 = jnp.dot(h2, w3_ref[...], preferred_element_type=jnp.float32) \
            + b3_ref[...]
        h3 = jnp.where((k >= 0) & (k < length), h3, 0.0)

        x1 = (jnp.dot(h3[0:W - 2], a1_ref[0], preferred_element_type=jnp.float32)
              + jnp.dot(h3[1:W - 1], a1_ref[1], preferred_element_type=jnp.float32)
              + jnp.dot(h3[2:W], a1_ref[2], preferred_element_type=jnp.float32)
              + bc1_ref[...])
        x1 = jnp.concatenate(
            [jnp.zeros((1, 4), jnp.float32), x1, jnp.zeros((1, 4), jnp.float32)],
            axis=0)
        x1 = jnp.where((k >= 0) & (k < maxl), x1, 0.0)

        res = (jnp.dot(x1[0:W - 2], a2_ref[0], preferred_element_type=jnp.float32)
               + jnp.dot(x1[1:W - 1], a2_ref[1], preferred_element_type=jnp.float32)
               + jnp.dot(x1[2:W], a2_ref[2], preferred_element_type=jnp.float32)
               + bc2_ref[...])
        vals = jnp.transpose(res[HALO - 1:HALO - 1 + T])
        jj = jax.lax.broadcasted_iota(jnp.int32, (1, T), 1)
        kq = tile0 + jj - start
        valid = (kq >= 0) & (kq < length)
        out_ref[...] = jnp.where(valid, vals, -jnp.inf)[None]


def kernel(sent_id, words, heads_ids, W1, b1, W2, b2, W3, b3, emb_table,
           Wc1, bc1, Wc2, bc2):
    heads_ids = heads_ids.astype(jnp.int32)
    head_sent = sent_id[heads_ids]
    starts = jnp.searchsorted(sent_id, head_sent, side='left').astype(jnp.int32)
    ends = jnp.searchsorted(sent_id, head_sent, side='right').astype(jnp.int32)
    lengths = ends - starts
    maxl = jnp.max(lengths).reshape(1)

    w1a = W1[:IN]
    w1b = W1[IN:2 * IN]
    w1c = W1[2 * IN:]

    wp = _wp_matmul(words, w1b)
    wpp = jnp.pad(wp, ((HALO, HALO), (0, 0)))
    hw = _hw_matmul(heads_ids, words.reshape(N_WORDS, 1, IN), w1a,
                    b1.reshape(1, HID))
    e1 = _e1_matmul(emb_table, w1c)

    a1 = jnp.transpose(Wc1, (2, 1, 0))
    a2 = jnp.transpose(Wc2, (2, 1, 0))

    grid_spec = pltpu.PrefetchScalarGridSpec(
        num_scalar_prefetch=4,
        grid=(N_HEADS, NT),
        in_specs=[
            pl.BlockSpec(memory_space=pl.ANY),
            pl.BlockSpec((1, 1, HID), lambda h, t, *_: (h, 0, 0)),
            pl.BlockSpec((MAXD, HID), lambda h, t, *_: (0, 0)),
            pl.BlockSpec((HID, MID), lambda h, t, *_: (0, 0)),
            pl.BlockSpec((1, MID), lambda h, t, *_: (0, 0)),
            pl.BlockSpec((MID, DE), lambda h, t, *_: (0, 0)),
            pl.BlockSpec((1, DE), lambda h, t, *_: (0, 0)),
            pl.BlockSpec((3, DE, 4), lambda h, t, *_: (0, 0, 0)),
            pl.BlockSpec((1, 4), lambda h, t, *_: (0, 0)),
            pl.BlockSpec((3, 4, 2), lambda h, t, *_: (0, 0, 0)),
            pl.BlockSpec((1, 2), lambda h, t, *_: (0, 0)),
        ],
        out_specs=pl.BlockSpec((1, 2, T), lambda h, t, *_: (h, 0, t)),
        scratch_shapes=[pltpu.VMEM((W, HID), jnp.float32),
                        pltpu.SemaphoreType.DMA],
    )
    out = pl.pallas_call(
        _span_kernel,
        grid_spec=grid_spec,
        out_shape=jax.ShapeDtypeStruct((N_HEADS, 2, N_WORDS), jnp.float32),
        compiler_params=pltpu.CompilerParams(
            dimension_semantics=("parallel", "arbitrary")),
    )(starts, lengths, heads_ids, maxl,
      wpp, hw, e1, W2,
      b2.reshape(1, MID), W3, b3.reshape(1, DE),
      a1, bc1.reshape(1, 4), a2, bc2.reshape(1, 2))
    return jnp.transpose(out, (0, 2, 1))
